# Initial kernel scaffold; baseline (speedup 1.0000x reference)
import jax
import jax.numpy as jnp
from jax.experimental import pallas as pl


def kernel(pc_source, pred_flow):
    raise NotImplementedError("write your pallas kernel here")


# speedup vs baseline: 24.0371x; 24.0371x over previous
import jax
import jax.numpy as jnp
from jax import lax
from jax.experimental import pallas as pl
from jax.experimental.pallas import tpu as pltpu

_K = 32
_R2 = 1.0
_BS_ITERS = 18
_QT = 256
_N = 4096
_B = 4


def _knn_loss_body(pc_ref, pcT_ref, fl_ref, flT_ref, out_ref):
    b = pl.program_id(0)
    qt = pl.program_id(1)
    qs = qt * _QT

    @pl.when((b == 0) & (qt == 0))
    def _init():
        out_ref[...] = jnp.zeros((1, 1), jnp.float32)

    xq = pc_ref[0, pl.ds(qs, _QT), 0:1]
    yq = pc_ref[0, pl.ds(qs, _QT), 1:2]
    zq = pc_ref[0, pl.ds(qs, _QT), 2:3]
    xm = pcT_ref[0, 0:1, :]
    ym = pcT_ref[0, 1:2, :]
    zm = pcT_ref[0, 2:3, :]

    dx = xq - xm
    dy = yq - ym
    dz = zq - zm
    D = dx * dx + dy * dy + dz * dz

    fxq = fl_ref[0, pl.ds(qs, _QT), 0:1]
    fyq = fl_ref[0, pl.ds(qs, _QT), 1:2]
    fzq = fl_ref[0, pl.ds(qs, _QT), 2:3]
    fxm = flT_ref[0, 0:1, :]
    fym = flT_ref[0, 1:2, :]
    fzm = flT_ref[0, 2:3, :]

    gx = fxq - fxm
    gy = fyq - fym
    gz = fzq - fzm
    sqF = gx * gx + gy * gy + gz * gz
    F = jnp.where(sqF > 0.0, jnp.sqrt(jnp.where(sqF > 0.0, sqF, 1.0)), 0.0)

    one = jnp.float32(1.0)
    zero = jnp.float32(0.0)
    kf = jnp.float32(_K)

    in_r = D <= _R2
    c_r = jnp.sum(jnp.where(in_r, one, zero), axis=1, keepdims=True)
    S_r = jnp.sum(jnp.where(in_r, F, zero), axis=1, keepdims=True)

    rowmin = jnp.min(D, axis=1, keepdims=True)
    colidx = lax.broadcasted_iota(jnp.int32, (_QT, _N), 1)
    cand = jnp.where(D == rowmin, colidx, jnp.int32(_N))
    jmin = jnp.min(cand, axis=1, keepdims=True)
    F0 = jnp.sum(jnp.where(cand == jmin, F, zero), axis=1, keepdims=True)

    lo = jnp.zeros((_QT, 1), jnp.float32)
    hi = jnp.ones((_QT, 1), jnp.float32)
    for _ in range(_BS_ITERS):
        mid = 0.5 * (lo + hi)
        cnt = jnp.sum(jnp.where(D <= mid, one, zero), axis=1, keepdims=True)
        ge = cnt >= kf
        hi = jnp.where(ge, mid, hi)
        lo = jnp.where(ge, lo, mid)

    m_lo = D <= lo
    m_hi = D <= hi
    c_lo = jnp.sum(jnp.where(m_lo, one, zero), axis=1, keepdims=True)
    S_lo = jnp.sum(jnp.where(m_lo, F, zero), axis=1, keepdims=True)
    c_hi = jnp.sum(jnp.where(m_hi, one, zero), axis=1, keepdims=True)
    S_hi = jnp.sum(jnp.where(m_hi, F, zero), axis=1, keepdims=True)

    denom = jnp.maximum(c_hi - c_lo, one)
    S_k = S_lo + (kf - c_lo) * (S_hi - S_lo) / denom

    rowsum = jnp.where(c_r >= kf, S_k, S_r + (kf - c_r) * F0)
    out_ref[...] += jnp.sum(rowsum, axis=(0, 1), keepdims=True)


def kernel(pc_source, pred_flow):
    pcT = jnp.transpose(pc_source, (0, 2, 1))
    flT = jnp.transpose(pred_flow, (0, 2, 1))
    total = pl.pallas_call(
        _knn_loss_body,
        grid=(_B, _N // _QT),
        in_specs=[
            pl.BlockSpec((1, _N, 3), lambda b, q: (b, 0, 0)),
            pl.BlockSpec((1, 3, _N), lambda b, q: (b, 0, 0)),
            pl.BlockSpec((1, _N, 3), lambda b, q: (b, 0, 0)),
            pl.BlockSpec((1, 3, _N), lambda b, q: (b, 0, 0)),
        ],
        out_specs=pl.BlockSpec((1, 1), lambda b, q: (0, 0)),
        out_shape=jax.ShapeDtypeStruct((1, 1), jnp.float32),
        compiler_params=pltpu.CompilerParams(
            dimension_semantics=("arbitrary", "arbitrary"),
        ),
    )(pc_source, pcT, pred_flow, flT)
    return total[0, 0] / jnp.float32(_B * _N * _K)


# speedup vs baseline: 27.3829x; 1.1392x over previous
import jax
import jax.numpy as jnp
from jax import lax
from jax.experimental import pallas as pl
from jax.experimental.pallas import tpu as pltpu

_K = 32
_R2 = 1.0
_BS_ITERS = 14
_QT = 256
_N = 4096
_B = 4


def _knn_loss_body(pc_ref, pcT_ref, fl_ref, flT_ref, out_ref):
    b = pl.program_id(0)
    qt = pl.program_id(1)
    qs = qt * _QT

    @pl.when((b == 0) & (qt == 0))
    def _init():
        out_ref[...] = jnp.zeros((1, 1), jnp.float32)

    xq = pc_ref[0, pl.ds(qs, _QT), 0:1]
    yq = pc_ref[0, pl.ds(qs, _QT), 1:2]
    zq = pc_ref[0, pl.ds(qs, _QT), 2:3]
    xm = pcT_ref[0, 0:1, :]
    ym = pcT_ref[0, 1:2, :]
    zm = pcT_ref[0, 2:3, :]

    dx = xq - xm
    dy = yq - ym
    dz = zq - zm
    D = dx * dx + dy * dy + dz * dz

    fxq = fl_ref[0, pl.ds(qs, _QT), 0:1]
    fyq = fl_ref[0, pl.ds(qs, _QT), 1:2]
    fzq = fl_ref[0, pl.ds(qs, _QT), 2:3]
    fxm = flT_ref[0, 0:1, :]
    fym = flT_ref[0, 1:2, :]
    fzm = flT_ref[0, 2:3, :]

    gx = fxq - fxm
    gy = fyq - fym
    gz = fzq - fzm
    sqF = gx * gx + gy * gy + gz * gz
    F = jnp.sqrt(sqF)

    one = jnp.float32(1.0)
    zero = jnp.float32(0.0)
    kf = jnp.float32(_K)

    in_r = D <= _R2
    c_r = jnp.sum(jnp.where(in_r, one, zero), axis=1, keepdims=True)
    S_r = jnp.sum(jnp.where(in_r, F, zero), axis=1, keepdims=True)

    rowmin = jnp.min(D, axis=1, keepdims=True)
    colidx = lax.broadcasted_iota(jnp.int32, (_QT, _N), 1)
    cand = jnp.where(D == rowmin, colidx, jnp.int32(_N))
    jmin = jnp.min(cand, axis=1, keepdims=True)
    F0 = jnp.sum(jnp.where(cand == jmin, F, zero), axis=1, keepdims=True)

    lo = jnp.zeros((_QT, 1), jnp.float32)
    hi = jnp.ones((_QT, 1), jnp.float32)
    for _ in range(_BS_ITERS):
        mid = 0.5 * (lo + hi)
        cnt = jnp.sum(jnp.where(D <= mid, one, zero), axis=1, keepdims=True)
        ge = cnt >= kf
        hi = jnp.where(ge, mid, hi)
        lo = jnp.where(ge, lo, mid)

    m_lo = D <= lo
    m_hi = D <= hi
    c_lo = jnp.sum(jnp.where(m_lo, one, zero), axis=1, keepdims=True)
    S_lo = jnp.sum(jnp.where(m_lo, F, zero), axis=1, keepdims=True)
    c_hi = jnp.sum(jnp.where(m_hi, one, zero), axis=1, keepdims=True)
    S_hi = jnp.sum(jnp.where(m_hi, F, zero), axis=1, keepdims=True)

    denom = jnp.maximum(c_hi - c_lo, one)
    S_k = S_lo + (kf - c_lo) * (S_hi - S_lo) / denom

    rowsum = jnp.where(c_r >= kf, S_k, S_r + (kf - c_r) * F0)
    out_ref[...] += jnp.sum(rowsum, axis=(0, 1), keepdims=True)


def kernel(pc_source, pred_flow):
    pcT = jnp.transpose(pc_source, (0, 2, 1))
    flT = jnp.transpose(pred_flow, (0, 2, 1))
    total = pl.pallas_call(
        _knn_loss_body,
        grid=(_B, _N // _QT),
        in_specs=[
            pl.BlockSpec((1, _N, 3), lambda b, q: (b, 0, 0)),
            pl.BlockSpec((1, 3, _N), lambda b, q: (b, 0, 0)),
            pl.BlockSpec((1, _N, 3), lambda b, q: (b, 0, 0)),
            pl.BlockSpec((1, 3, _N), lambda b, q: (b, 0, 0)),
        ],
        out_specs=pl.BlockSpec((1, 1), lambda b, q: (0, 0)),
        out_shape=jax.ShapeDtypeStruct((1, 1), jnp.float32),
        compiler_params=pltpu.CompilerParams(
            dimension_semantics=("arbitrary", "arbitrary"),
        ),
    )(pc_source, pcT, pred_flow, flT)
    return total[0, 0] / jnp.float32(_B * _N * _K)


# speedup vs baseline: 31.8743x; 1.1640x over previous
import jax
import jax.numpy as jnp
from jax import lax
from jax.experimental import pallas as pl
from jax.experimental.pallas import tpu as pltpu

_K = 32
_R2 = 1.0
_BS_ITERS = 14
_QT = 256
_N = 4096
_B = 4
_CPAD = 8


def _knn_loss_body(pc_ref, pcT_ref, fl_ref, flT_ref, out_ref):
    b = pl.program_id(0)
    qt = pl.program_id(1)
    qs = qt * _QT

    @pl.when((b == 0) & (qt == 0))
    def _init():
        out_ref[...] = jnp.zeros((1, 1), jnp.float32)

    one = jnp.float32(1.0)
    zero = jnp.float32(0.0)
    kf = jnp.float32(_K)

    aq = pc_ref[0, pl.ds(qs, _QT), :]
    bm = pcT_ref[0]
    sqq = jnp.sum(aq * aq, axis=1, keepdims=True)
    sqm = jnp.sum(bm * bm, axis=0, keepdims=True)
    dotD = jnp.dot(aq, bm, preferred_element_type=jnp.float32)
    D = (sqq + sqm) - (dotD + dotD)

    fq = fl_ref[0, pl.ds(qs, _QT), :]
    fm = flT_ref[0]
    fsqq = jnp.sum(fq * fq, axis=1, keepdims=True)
    fsqm = jnp.sum(fm * fm, axis=0, keepdims=True)
    dotF = jnp.dot(fq, fm, preferred_element_type=jnp.float32)
    sqF = jnp.maximum((fsqq + fsqm) - (dotF + dotF), zero)
    F = jnp.sqrt(sqF)

    rowmin = jnp.min(D, axis=1, keepdims=True)
    colidx = lax.broadcasted_iota(jnp.int32, (_QT, _N), 1)
    cand = jnp.where(D == rowmin, colidx, jnp.int32(_N))
    jmin = jnp.min(cand, axis=1, keepdims=True)
    F0 = jnp.sum(jnp.where(cand == jmin, F, zero), axis=1, keepdims=True)

    lo = jnp.zeros((_QT, 1), jnp.float32)
    hi = jnp.ones((_QT, 1), jnp.float32)
    for _ in range(_BS_ITERS):
        mid = 0.5 * (lo + hi)
        cnt = jnp.sum(jnp.where(D <= mid, one, zero), axis=1, keepdims=True)
        ge = cnt >= kf
        hi = jnp.where(ge, mid, hi)
        lo = jnp.where(ge, lo, mid)

    m_lo = D <= lo
    m_hi = D <= hi
    c_lo = jnp.sum(jnp.where(m_lo, one, zero), axis=1, keepdims=True)
    S_lo = jnp.sum(jnp.where(m_lo, F, zero), axis=1, keepdims=True)
    c_hi = jnp.sum(jnp.where(m_hi, one, zero), axis=1, keepdims=True)
    S_hi = jnp.sum(jnp.where(m_hi, F, zero), axis=1, keepdims=True)

    denom = jnp.maximum(c_hi - c_lo, one)
    S_k = S_lo + (kf - c_lo) * (S_hi - S_lo) / denom

    rowsum = jnp.where(c_hi >= kf, S_k, S_hi + (kf - c_hi) * F0)
    out_ref[...] += jnp.sum(rowsum, axis=(0, 1), keepdims=True)


def kernel(pc_source, pred_flow):
    pad = ((0, 0), (0, 0), (0, _CPAD - 3))
    pc = jnp.pad(pc_source, pad)
    fl = jnp.pad(pred_flow, pad)
    pcT = jnp.transpose(pc, (0, 2, 1))
    flT = jnp.transpose(fl, (0, 2, 1))
    total = pl.pallas_call(
        _knn_loss_body,
        grid=(_B, _N // _QT),
        in_specs=[
            pl.BlockSpec((1, _N, _CPAD), lambda b, q: (b, 0, 0)),
            pl.BlockSpec((1, _CPAD, _N), lambda b, q: (b, 0, 0)),
            pl.BlockSpec((1, _N, _CPAD), lambda b, q: (b, 0, 0)),
            pl.BlockSpec((1, _CPAD, _N), lambda b, q: (b, 0, 0)),
        ],
        out_specs=pl.BlockSpec((1, 1), lambda b, q: (0, 0)),
        out_shape=jax.ShapeDtypeStruct((1, 1), jnp.float32),
        compiler_params=pltpu.CompilerParams(
            dimension_semantics=("arbitrary", "arbitrary"),
        ),
    )(pc, pcT, fl, flT)
    return total[0, 0] / jnp.float32(_B * _N * _K)


# speedup vs baseline: 33.7961x; 1.0603x over previous
import jax
import jax.numpy as jnp
from jax import lax
from jax.experimental import pallas as pl
from jax.experimental.pallas import tpu as pltpu

_K = 32
_R2 = 1.0
_BS_ITERS = 8
_QT = 256
_N = 4096
_B = 4
_CPAD = 8


def _knn_loss_body(pc_ref, pcT_ref, fl_ref, flT_ref, out_ref):
    b = pl.program_id(0)
    qt = pl.program_id(1)
    qs = qt * _QT

    @pl.when((b == 0) & (qt == 0))
    def _init():
        out_ref[...] = jnp.zeros((1, 1), jnp.float32)

    one = jnp.float32(1.0)
    zero = jnp.float32(0.0)
    kf = jnp.float32(_K)

    aq = pc_ref[0, pl.ds(qs, _QT), :]
    bm = pcT_ref[0]
    sqq = jnp.sum(aq * aq, axis=1, keepdims=True)
    sqm = jnp.sum(bm * bm, axis=0, keepdims=True)
    dotD = jnp.dot(aq, bm, preferred_element_type=jnp.float32)
    D = (sqq + sqm) - (dotD + dotD)

    fq = fl_ref[0, pl.ds(qs, _QT), :]
    fm = flT_ref[0]
    fsqq = jnp.sum(fq * fq, axis=1, keepdims=True)
    fsqm = jnp.sum(fm * fm, axis=0, keepdims=True)
    dotF = jnp.dot(fq, fm, preferred_element_type=jnp.float32)
    sqF = jnp.maximum((fsqq + fsqm) - (dotF + dotF), zero)
    F = jnp.sqrt(sqF)

    rowmin = jnp.min(D, axis=1, keepdims=True)
    colidx = lax.broadcasted_iota(jnp.int32, (_QT, _N), 1)
    cand = jnp.where(D == rowmin, colidx, jnp.int32(_N))
    jmin = jnp.min(cand, axis=1, keepdims=True)
    F0 = jnp.sum(jnp.where(cand == jmin, F, zero), axis=1, keepdims=True)

    kt = jnp.float32(_K) - 0.5
    lo = jnp.zeros((_QT, 1), jnp.float32)
    hi = jnp.ones((_QT, 1), jnp.float32)
    c_lo_v = jnp.zeros((_QT, 1), jnp.float32)
    c_hi_v = jnp.sum(jnp.where(D <= one, one, zero), axis=1, keepdims=True)
    last_ge = jnp.zeros((_QT, 1), jnp.bool_)
    for it in range(_BS_ITERS):
        u_lo = lo * jnp.sqrt(lo)
        u_hi = hi * jnp.sqrt(hi)
        den = jnp.maximum(c_hi_v - c_lo_v, jnp.float32(1e-6))
        u_mid = jnp.maximum(u_lo + (u_hi - u_lo) * (kt - c_lo_v) / den, zero)
        mid = jnp.exp(jnp.log(jnp.maximum(u_mid, jnp.float32(1e-30)))
                      * jnp.float32(2.0 / 3.0))
        span = hi - lo
        mid = jnp.clip(mid, lo + 0.02 * span, hi - 0.02 * span)
        cnt = jnp.sum(jnp.where(D <= mid, one, zero), axis=1, keepdims=True)
        ge = cnt >= kf
        if it > 0:
            stag_lo = ge & last_ge
            stag_hi = (~ge) & (~last_ge)
            c_lo_v = jnp.where(stag_lo, kt - 0.5 * (kt - c_lo_v), c_lo_v)
            c_hi_v = jnp.where(stag_hi, kt + 0.5 * (c_hi_v - kt), c_hi_v)
        hi, lo = jnp.where(ge, mid, hi), jnp.where(ge, lo, mid)
        c_hi_v = jnp.where(ge, cnt, c_hi_v)
        c_lo_v = jnp.where(ge, c_lo_v, cnt)
        last_ge = ge

    m_lo = D <= lo
    m_hi = D <= hi
    c_lo = jnp.sum(jnp.where(m_lo, one, zero), axis=1, keepdims=True)
    S_lo = jnp.sum(jnp.where(m_lo, F, zero), axis=1, keepdims=True)
    c_hi = jnp.sum(jnp.where(m_hi, one, zero), axis=1, keepdims=True)
    S_hi = jnp.sum(jnp.where(m_hi, F, zero), axis=1, keepdims=True)

    denom = jnp.maximum(c_hi - c_lo, one)
    S_k = S_lo + (kf - c_lo) * (S_hi - S_lo) / denom

    rowsum = jnp.where(c_hi >= kf, S_k, S_hi + (kf - c_hi) * F0)
    out_ref[...] += jnp.sum(rowsum, axis=(0, 1), keepdims=True)


def kernel(pc_source, pred_flow):
    pad = ((0, 0), (0, 0), (0, _CPAD - 3))
    pc = jnp.pad(pc_source, pad)
    fl = jnp.pad(pred_flow, pad)
    pcT = jnp.transpose(pc, (0, 2, 1))
    flT = jnp.transpose(fl, (0, 2, 1))
    total = pl.pallas_call(
        _knn_loss_body,
        grid=(_B, _N // _QT),
        in_specs=[
            pl.BlockSpec((1, _N, _CPAD), lambda b, q: (b, 0, 0)),
            pl.BlockSpec((1, _CPAD, _N), lambda b, q: (b, 0, 0)),
            pl.BlockSpec((1, _N, _CPAD), lambda b, q: (b, 0, 0)),
            pl.BlockSpec((1, _CPAD, _N), lambda b, q: (b, 0, 0)),
        ],
        out_specs=pl.BlockSpec((1, 1), lambda b, q: (0, 0)),
        out_shape=jax.ShapeDtypeStruct((1, 1), jnp.float32),
        compiler_params=pltpu.CompilerParams(
            dimension_semantics=("arbitrary", "arbitrary"),
        ),
    )(pc, pcT, fl, flT)
    return total[0, 0] / jnp.float32(_B * _N * _K)


# speedup vs baseline: 37.5613x; 1.1114x over previous
import jax
import jax.numpy as jnp
from jax import lax
from jax.experimental import pallas as pl
from jax.experimental.pallas import tpu as pltpu

_K = 32
_R2 = 1.0
_BS_ITERS = 8
_QT = 256
_N = 4096
_B = 4
_CPAD = 8


def _knn_loss_body(pc_ref, pcT_ref, fl_ref, flT_ref, out_ref):
    b = pl.program_id(0)
    qt = pl.program_id(1)
    qs = qt * _QT

    @pl.when((b == 0) & (qt == 0))
    def _init():
        out_ref[...] = jnp.zeros((1, 1), jnp.float32)

    one = jnp.float32(1.0)
    zero = jnp.float32(0.0)
    kf = jnp.float32(_K)

    aq = pc_ref[0, pl.ds(qs, _QT), :]
    bm = pcT_ref[0]
    sqq = jnp.sum(aq * aq, axis=1, keepdims=True)
    sqm = jnp.sum(bm * bm, axis=0, keepdims=True)
    dotD = jnp.dot(aq, bm, preferred_element_type=jnp.float32)
    D = (sqq + sqm) - (dotD + dotD)

    fq = fl_ref[0, pl.ds(qs, _QT), :]
    fm = flT_ref[0]
    fsqq = jnp.sum(fq * fq, axis=1, keepdims=True)
    fsqm = jnp.sum(fm * fm, axis=0, keepdims=True)
    dotF = jnp.dot(fq, fm, preferred_element_type=jnp.float32)
    sqF = jnp.maximum((fsqq + fsqm) - (dotF + dotF), zero)
    F = jnp.sqrt(sqF)


    kt = jnp.float32(_K) - 0.5
    lo = jnp.zeros((_QT, 1), jnp.float32)
    hi = jnp.ones((_QT, 1), jnp.float32)
    c_lo_v = jnp.zeros((_QT, 1), jnp.float32)
    c_hi_v = jnp.sum(jnp.where(D <= one, one, zero), axis=1, keepdims=True)
    last_ge = jnp.zeros((_QT, 1), jnp.bool_)
    for it in range(_BS_ITERS):
        u_lo = lo * jnp.sqrt(lo)
        u_hi = hi * jnp.sqrt(hi)
        den = jnp.maximum(c_hi_v - c_lo_v, jnp.float32(1e-6))
        u_mid = jnp.maximum(u_lo + (u_hi - u_lo) * (kt - c_lo_v) / den, zero)
        mid = jnp.exp(jnp.log(jnp.maximum(u_mid, jnp.float32(1e-30)))
                      * jnp.float32(2.0 / 3.0))
        span = hi - lo
        mid = jnp.clip(mid, lo + 0.02 * span, hi - 0.02 * span)
        cnt = jnp.sum(jnp.where(D <= mid, one, zero), axis=1, keepdims=True)
        ge = cnt >= kf
        if it > 0:
            stag_lo = ge & last_ge
            stag_hi = (~ge) & (~last_ge)
            c_lo_v = jnp.where(stag_lo, kt - 0.5 * (kt - c_lo_v), c_lo_v)
            c_hi_v = jnp.where(stag_hi, kt + 0.5 * (c_hi_v - kt), c_hi_v)
        hi, lo = jnp.where(ge, mid, hi), jnp.where(ge, lo, mid)
        c_hi_v = jnp.where(ge, cnt, c_hi_v)
        c_lo_v = jnp.where(ge, c_lo_v, cnt)
        last_ge = ge

    m_lo = D <= lo
    m_hi = D <= hi
    c_lo = jnp.sum(jnp.where(m_lo, one, zero), axis=1, keepdims=True)
    S_lo = jnp.sum(jnp.where(m_lo, F, zero), axis=1, keepdims=True)
    c_hi = jnp.sum(jnp.where(m_hi, one, zero), axis=1, keepdims=True)
    S_hi = jnp.sum(jnp.where(m_hi, F, zero), axis=1, keepdims=True)

    denom = jnp.maximum(c_hi - c_lo, one)
    S_k = S_lo + (kf - c_lo) * (S_hi - S_lo) / denom

    rowsum = jnp.where(c_hi >= kf, S_k, S_hi)
    out_ref[...] += jnp.sum(rowsum, axis=(0, 1), keepdims=True)


def kernel(pc_source, pred_flow):
    pad = ((0, 0), (0, 0), (0, _CPAD - 3))
    pc = jnp.pad(pc_source, pad)
    fl = jnp.pad(pred_flow, pad)
    pcT = jnp.transpose(pc, (0, 2, 1))
    flT = jnp.transpose(fl, (0, 2, 1))
    total = pl.pallas_call(
        _knn_loss_body,
        grid=(_B, _N // _QT),
        in_specs=[
            pl.BlockSpec((1, _N, _CPAD), lambda b, q: (b, 0, 0)),
            pl.BlockSpec((1, _CPAD, _N), lambda b, q: (b, 0, 0)),
            pl.BlockSpec((1, _N, _CPAD), lambda b, q: (b, 0, 0)),
            pl.BlockSpec((1, _CPAD, _N), lambda b, q: (b, 0, 0)),
        ],
        out_specs=pl.BlockSpec((1, 1), lambda b, q: (0, 0)),
        out_shape=jax.ShapeDtypeStruct((1, 1), jnp.float32),
        compiler_params=pltpu.CompilerParams(
            dimension_semantics=("arbitrary", "arbitrary"),
        ),
    )(pc, pcT, fl, flT)
    return total[0, 0] / jnp.float32(_B * _N * _K)


# speedup vs baseline: 37.6893x; 1.0034x over previous
import jax
import jax.numpy as jnp
from jax import lax
from jax.experimental import pallas as pl
from jax.experimental.pallas import tpu as pltpu

_K = 32
_R2 = 1.0
_BS_ITERS = 8
_QT = 512
_N = 4096
_B = 4
_CPAD = 8


def _knn_loss_body(pc_ref, pcT_ref, fl_ref, flT_ref, out_ref):
    b = pl.program_id(0)
    qt = pl.program_id(1)
    qs = qt * _QT

    @pl.when((b == 0) & (qt == 0))
    def _init():
        out_ref[...] = jnp.zeros((1, 1), jnp.float32)

    one = jnp.float32(1.0)
    zero = jnp.float32(0.0)
    kf = jnp.float32(_K)

    aq = pc_ref[0, pl.ds(qs, _QT), :]
    bm = pcT_ref[0]
    sqq = jnp.sum(aq * aq, axis=1, keepdims=True)
    sqm = jnp.sum(bm * bm, axis=0, keepdims=True)
    dotD = jnp.dot(aq, bm, preferred_element_type=jnp.float32)
    D = (sqq + sqm) - (dotD + dotD)

    fq = fl_ref[0, pl.ds(qs, _QT), :]
    fm = flT_ref[0]
    fsqq = jnp.sum(fq * fq, axis=1, keepdims=True)
    fsqm = jnp.sum(fm * fm, axis=0, keepdims=True)
    dotF = jnp.dot(fq, fm, preferred_element_type=jnp.float32)
    sqF = jnp.maximum((fsqq + fsqm) - (dotF + dotF), zero)
    F = jnp.sqrt(sqF)


    kt = jnp.float32(_K) - 0.5
    lo = jnp.zeros((_QT, 1), jnp.float32)
    hi = jnp.ones((_QT, 1), jnp.float32)
    c_lo_v = jnp.zeros((_QT, 1), jnp.float32)
    c_hi_v = jnp.sum(jnp.where(D <= one, one, zero), axis=1, keepdims=True)
    last_ge = jnp.zeros((_QT, 1), jnp.bool_)
    for it in range(_BS_ITERS):
        u_lo = lo * jnp.sqrt(lo)
        u_hi = hi * jnp.sqrt(hi)
        den = jnp.maximum(c_hi_v - c_lo_v, jnp.float32(1e-6))
        u_mid = jnp.maximum(u_lo + (u_hi - u_lo) * (kt - c_lo_v) / den, zero)
        mid = jnp.exp(jnp.log(jnp.maximum(u_mid, jnp.float32(1e-30)))
                      * jnp.float32(2.0 / 3.0))
        span = hi - lo
        mid = jnp.clip(mid, lo + 0.02 * span, hi - 0.02 * span)
        cnt = jnp.sum(jnp.where(D <= mid, one, zero), axis=1, keepdims=True)
        ge = cnt >= kf
        if it > 0:
            stag_lo = ge & last_ge
            stag_hi = (~ge) & (~last_ge)
            c_lo_v = jnp.where(stag_lo, kt - 0.5 * (kt - c_lo_v), c_lo_v)
            c_hi_v = jnp.where(stag_hi, kt + 0.5 * (c_hi_v - kt), c_hi_v)
        hi, lo = jnp.where(ge, mid, hi), jnp.where(ge, lo, mid)
        c_hi_v = jnp.where(ge, cnt, c_hi_v)
        c_lo_v = jnp.where(ge, c_lo_v, cnt)
        last_ge = ge

    m_lo = D <= lo
    m_hi = D <= hi
    c_lo = jnp.sum(jnp.where(m_lo, one, zero), axis=1, keepdims=True)
    S_lo = jnp.sum(jnp.where(m_lo, F, zero), axis=1, keepdims=True)
    c_hi = jnp.sum(jnp.where(m_hi, one, zero), axis=1, keepdims=True)
    S_hi = jnp.sum(jnp.where(m_hi, F, zero), axis=1, keepdims=True)

    denom = jnp.maximum(c_hi - c_lo, one)
    S_k = S_lo + (kf - c_lo) * (S_hi - S_lo) / denom

    rowsum = jnp.where(c_hi >= kf, S_k, S_hi)
    out_ref[...] += jnp.sum(rowsum, axis=(0, 1), keepdims=True)


def kernel(pc_source, pred_flow):
    pad = ((0, 0), (0, 0), (0, _CPAD - 3))
    pc = jnp.pad(pc_source, pad)
    fl = jnp.pad(pred_flow, pad)
    pcT = jnp.transpose(pc, (0, 2, 1))
    flT = jnp.transpose(fl, (0, 2, 1))
    total = pl.pallas_call(
        _knn_loss_body,
        grid=(_B, _N // _QT),
        in_specs=[
            pl.BlockSpec((1, _N, _CPAD), lambda b, q: (b, 0, 0)),
            pl.BlockSpec((1, _CPAD, _N), lambda b, q: (b, 0, 0)),
            pl.BlockSpec((1, _N, _CPAD), lambda b, q: (b, 0, 0)),
            pl.BlockSpec((1, _CPAD, _N), lambda b, q: (b, 0, 0)),
        ],
        out_specs=pl.BlockSpec((1, 1), lambda b, q: (0, 0)),
        out_shape=jax.ShapeDtypeStruct((1, 1), jnp.float32),
        compiler_params=pltpu.CompilerParams(
            dimension_semantics=("arbitrary", "arbitrary"),
        ),
    )(pc, pcT, fl, flT)
    return total[0, 0] / jnp.float32(_B * _N * _K)
